# Initial kernel scaffold; baseline (speedup 1.0000x reference)
import jax
import jax.numpy as jnp
from jax.experimental import pallas as pl


def kernel(x, edge_index, W1, b1, W2, b2, W3, b3, g1, be1, g2, be2):
    raise NotImplementedError("write your pallas kernel here")


# speedup vs baseline: 22.6881x; 22.6881x over previous
import functools

import jax
import jax.numpy as jnp
from jax import lax
from jax.experimental import pallas as pl
from jax.experimental.pallas import tpu as pltpu
from jax.experimental.pallas import tpu_sc as plsc

NC = 2
NS = 16
BLK = 128

_HIGH = jax.lax.Precision.HIGHEST


def _mesh():
    return plsc.VectorSubcoreMesh(
        core_axis_name="c", subcore_axis_name="s", num_cores=NC, num_subcores=NS
    )


def _fill_const(ref, n_rows, f, val):
    vec = jnp.full((16,), val, jnp.float32)

    def row(i, c):
        for j in range(f // 16):
            ref[i, pl.ds(j * 16, 16)] = vec
        return c

    lax.fori_loop(0, n_rows, row, 0)


def _make_deg_pass(n_blk, nacc):
    f = 16
    rpt = nacc // NS

    @functools.partial(
        pl.kernel,
        out_type=jax.ShapeDtypeStruct((NC, nacc, f), jnp.float32),
        mesh=_mesh(),
        compiler_params=pltpu.CompilerParams(use_tc_tiling_on_sc=False),
        scratch_types=[
            pltpu.VMEM((n_blk, BLK), jnp.int32),
            pltpu.VMEM((BLK, f), jnp.float32),
            pltpu.VMEM_SHARED((nacc, f), jnp.float32),
        ],
    )
    def kern(dst_hbm, out_hbm, dst_v, ones_v, acc_sh):
        cid = lax.axis_index("c")
        sid = lax.axis_index("s")
        wid = cid * NS + sid
        _fill_const(ones_v, BLK, f, 0.0)
        for z in range(rpt // BLK):
            pltpu.sync_copy(ones_v, acc_sh.at[pl.ds(sid * rpt + z * BLK, BLK)])
        _fill_const(ones_v, BLK, f, 1.0)
        pltpu.sync_copy(dst_hbm.at[wid], dst_v)
        plsc.subcore_barrier()

        def body(j, c):
            pltpu.sync_copy(ones_v, acc_sh.at[dst_v.at[j]], add=True)
            return c

        lax.fori_loop(0, n_blk, body, 0)
        plsc.subcore_barrier()
        pltpu.sync_copy(
            acc_sh.at[pl.ds(sid * rpt, rpt)], out_hbm.at[cid, pl.ds(sid * rpt, rpt)]
        )

    return kern


def _make_agg_pass(n_blk, nacc, f):
    rpt = nacc // NS

    @functools.partial(
        pl.kernel,
        out_type=jax.ShapeDtypeStruct((NC, nacc, f), jnp.float32),
        mesh=_mesh(),
        compiler_params=pltpu.CompilerParams(use_tc_tiling_on_sc=False),
        scratch_types=[
            pltpu.VMEM((n_blk, BLK), jnp.int32),
            pltpu.VMEM((n_blk, BLK), jnp.int32),
            pltpu.VMEM((BLK, f), jnp.float32),
            pltpu.VMEM((BLK, f), jnp.float32),
            pltpu.VMEM_SHARED((nacc, f), jnp.float32),
            pltpu.SemaphoreType.DMA,
            pltpu.SemaphoreType.DMA,
        ],
    )
    def kern(src_hbm, dst_hbm, table_hbm, out_hbm,
             src_v, dst_v, rows_a, rows_b, acc_sh, sem_a, sem_b):
        cid = lax.axis_index("c")
        sid = lax.axis_index("s")
        wid = cid * NS + sid
        _fill_const(rows_a, BLK, f, 0.0)
        for z in range(rpt // BLK):
            pltpu.sync_copy(rows_a, acc_sh.at[pl.ds(sid * rpt + z * BLK, BLK)])
        pltpu.sync_copy(src_hbm.at[wid], src_v)
        pltpu.sync_copy(dst_hbm.at[wid], dst_v)
        plsc.subcore_barrier()

        def body(i, c):
            j0 = i * 2
            ca = pltpu.async_copy(table_hbm.at[src_v.at[j0]], rows_a, sem_a)
            cb = pltpu.async_copy(table_hbm.at[src_v.at[j0 + 1]], rows_b, sem_b)
            ca.wait()
            pltpu.sync_copy(rows_a, acc_sh.at[dst_v.at[j0]], add=True)
            cb.wait()
            pltpu.sync_copy(rows_b, acc_sh.at[dst_v.at[j0 + 1]], add=True)
            return c

        lax.fori_loop(0, n_blk // 2, body, 0)
        if n_blk % 2:
            j0 = n_blk - 1
            pltpu.async_copy(table_hbm.at[src_v.at[j0]], rows_a, sem_a).wait()
            pltpu.sync_copy(rows_a, acc_sh.at[dst_v.at[j0]], add=True)
        plsc.subcore_barrier()
        pltpu.sync_copy(
            acc_sh.at[pl.ds(sid * rpt, rpt)], out_hbm.at[cid, pl.ds(sid * rpt, rpt)]
        )

    return kern


def _dinv_of(degp_ref, n):
    deg = degp_ref[0, 0:n, :] + degp_ref[1, 0:n, :] + 1.0
    return 1.0 / jnp.sqrt(deg[:, 0:1])


def _mm1_body(n, x_ref, w_ref, out_ref):
    out_ref[...] = jnp.dot(x_ref[...], w_ref[...], precision=_HIGH,
                           preferred_element_type=jnp.float32)


def _scale_body(n, xt_ref, degp_ref, out_ref):
    out_ref[...] = xt_ref[...] * _dinv_of(degp_ref, n)


def _post1_body(n, p_ref, xs_ref, degp_ref, b_ref, g_ref, be_ref, w_ref, out_ref):
    dinv = _dinv_of(degp_ref, n)
    acc = p_ref[0, 0:n, :] + p_ref[1, 0:n, :] + xs_ref[...]
    h = jnp.maximum(acc * dinv + b_ref[...], 0.0)
    m = jnp.mean(h, axis=0, keepdims=True)
    v = jnp.mean((h - m) ** 2, axis=0, keepdims=True)
    hbn = (h - m) * (1.0 / jnp.sqrt(v + 1e-5)) * g_ref[...] + be_ref[...]
    xt = jnp.dot(hbn, w_ref[...], precision=_HIGH, preferred_element_type=jnp.float32)
    out_ref[...] = xt * dinv


def _post2_body(n, p_ref, xs_ref, degp_ref, b_ref, g_ref, be_ref, w_ref,
                x1_ref, xs3_ref):
    dinv = _dinv_of(degp_ref, n)
    acc = p_ref[0, 0:n, :] + p_ref[1, 0:n, :] + xs_ref[...]
    x1f = acc * dinv + b_ref[...]
    x1 = x1f[:, 0:8]
    x1_ref[...] = x1
    x2 = jnp.maximum(x1, 0.0)
    m = jnp.mean(x2, axis=0, keepdims=True)
    v = jnp.mean((x2 - m) ** 2, axis=0, keepdims=True)
    xbn = (x2 - m) * (1.0 / jnp.sqrt(v + 1e-5)) * g_ref[...] + be_ref[...]
    xt = jnp.dot(xbn, w_ref[...], precision=_HIGH, preferred_element_type=jnp.float32)
    xs3_ref[...] = xt * dinv


def _post3_body(n, p_ref, xs_ref, degp_ref, b_ref, out_ref):
    dinv = _dinv_of(degp_ref, n)
    acc = p_ref[0, 0:n, :] + p_ref[1, 0:n, :] + xs_ref[...]
    out_ref[...] = acc * dinv + b_ref[...]


def _tc(body, out_shape, *args):
    return pl.pallas_call(
        body, out_shape=out_shape,
        compiler_params=pltpu.CompilerParams(vmem_limit_bytes=120 * 1024 * 1024),
    )(*args)


def kernel(x, edge_index, W1, b1, W2, b2, W3, b3, g1, be1, g2, be2):
    n = x.shape[-2]
    e = edge_index.shape[1]
    f1 = W1.shape[1]
    x2d = x.reshape(n, x.shape[-1])

    t = NC * NS
    n_blk = -(-e // (t * BLK))
    ep = t * n_blk * BLK
    nacc = -(-(n + 1) // (NS * BLK)) * (NS * BLK)

    pad = ep - e
    src_p = jnp.concatenate(
        [edge_index[0], jnp.zeros((pad,), jnp.int32)]).reshape(t, n_blk, BLK)
    dst_p = jnp.concatenate(
        [edge_index[1], jnp.full((pad,), n, jnp.int32)]).reshape(t, n_blk, BLK)

    w2p = jnp.zeros((f1, 16), jnp.float32).at[:, :8].set(W2)
    b2p = jnp.zeros((16,), jnp.float32).at[:8].set(b2)
    w3p = jnp.zeros((8, 16), jnp.float32).at[:, :3].set(W3)
    b3p = jnp.zeros((16,), jnp.float32).at[:3].set(b3)

    deg_pass = _make_deg_pass(n_blk, nacc)
    agg64 = _make_agg_pass(n_blk, nacc, f1)
    agg16 = _make_agg_pass(n_blk, nacc, 16)

    degp = deg_pass(dst_p)
    xt1 = _tc(functools.partial(_mm1_body, n),
              jax.ShapeDtypeStruct((n, f1), jnp.float32), x2d, W1)
    xs1 = _tc(functools.partial(_scale_body, n),
              jax.ShapeDtypeStruct((n, f1), jnp.float32), xt1, degp)
    p1 = agg64(src_p, dst_p, xs1)
    xs2 = _tc(functools.partial(_post1_body, n),
              jax.ShapeDtypeStruct((n, 16), jnp.float32),
              p1, xs1, degp, b1, g1, be1, w2p)
    p2 = agg16(src_p, dst_p, xs2)
    x1, xs3 = _tc(functools.partial(_post2_body, n),
                  [jax.ShapeDtypeStruct((n, 8), jnp.float32),
                   jax.ShapeDtypeStruct((n, 16), jnp.float32)],
                  p2, xs2, degp, b2p, g2, be2, w3p)
    p3 = agg16(src_p, dst_p, xs3)
    out16 = _tc(functools.partial(_post3_body, n),
                jax.ShapeDtypeStruct((n, 16), jnp.float32),
                p3, xs3, degp, b3p)

    out = out16[:, :3].reshape(1, 1, n, 3)
    return (out, x1.reshape(1, 1, n, 8))
